# Initial kernel scaffold; baseline (speedup 1.0000x reference)
import jax
import jax.numpy as jnp
from jax.experimental import pallas as pl


def kernel(coordinates, w):
    raise NotImplementedError("write your pallas kernel here")


# speedup vs baseline: 2.3642x; 2.3642x over previous
import functools

import jax
import jax.numpy as jnp
from jax import lax
from jax.experimental import pallas as pl
from jax.experimental.pallas import tpu as pltpu
from jax.experimental.pallas import tpu_sc as plsc

_BINWIDTHS = (100, 200, 500, 1000, 2000, 5000)
_WINDOW = (-100000, 100000)
_NDIM = 100
_LANES = 16
_DPAD = 128
_NV = _DPAD // _LANES
_F = 32768
_C = 2
_NC, _NS = 2, 16
_NW = _NC * _NS
_FPW = _F // _NW
_CF = 16
_NCHUNK = _FPW // _CF
_NTERMS = 2 * _C * len(_BINWIDTHS)


def _row_offsets():
    offs, start = [], 0
    for b in _BINWIDTHS:
        nb = (_WINDOW[1] - _WINDOW[0]) // b + 1
        offs.append(start - (_WINDOW[0] // b))
        start += nb
    return tuple(offs), start


_OFFS, _NROWS = _row_offsets()


def _sc_body(coords_hbm, w_hbm, out_hbm, coords_v, rows_v, wbuf, outbuf, sem):
    wid = lax.axis_index("s") * _NC + lax.axis_index("c")
    base = wid * _FPW
    pltpu.sync_copy(coords_hbm.at[pl.ds(base, _FPW)], coords_v.at[0])
    pltpu.sync_copy(coords_hbm.at[pl.ds(_F + base, _FPW)], coords_v.at[1])

    def chunk_body(g, carry):
        cvecs = [coords_v[ci, pl.ds(g * _CF, _CF)] for ci in range(_C)]
        copies = []
        kk = 0
        for b, off in zip(_BINWIDTHS, _OFFS):
            inv = jnp.float32(1.0 / b)
            for c in cvecs:
                q = ((c.astype(jnp.float32) + 0.5) * inv).astype(jnp.int32)
                r = c - q * b
                alpha = r.astype(jnp.float32) * inv
                i0 = q + off
                wbuf[kk] = 1.0 - alpha
                wbuf[kk + 1] = alpha
                copies.append(pltpu.async_copy(
                    w_hbm.at[i0], rows_v.at[pl.ds(kk * _CF, _CF)], sem))
                copies.append(pltpu.async_copy(
                    w_hbm.at[i0 + 1], rows_v.at[pl.ds((kk + 1) * _CF, _CF)], sem))
                kk += 2
        for cp in copies:
            cp.wait()

        lane = lax.iota(jnp.int32, _LANES)

        def d_body(d, c2):
            dcol = jnp.full((_LANES,), d, jnp.int32)
            acc = jnp.zeros((_LANES,), jnp.float32)
            for k in range(_NTERMS):
                rowidx = k * _CF + lane
                acc = acc + plsc.load_gather(rows_v, [rowidx, dcol]) * wbuf[k]
            plsc.store_scatter(outbuf, [lane, dcol], acc)
            return c2

        lax.fori_loop(0, _NDIM, d_body, 0)
        pltpu.sync_copy(outbuf, out_hbm.at[pl.ds(base + g * _CF, _CF)])
        return carry

    lax.fori_loop(0, _NCHUNK, chunk_body, 0)


_launch = functools.partial(
    pl.kernel,
    out_type=jax.ShapeDtypeStruct((_F, _DPAD), jnp.float32),
    scratch_types=[
        pltpu.VMEM((_C, _FPW), jnp.int32),
        pltpu.VMEM((_NTERMS * _CF, _DPAD), jnp.float32),
        pltpu.VMEM((_NTERMS, _CF), jnp.float32),
        pltpu.VMEM((_CF, _DPAD), jnp.float32),
        pltpu.SemaphoreType.DMA,
    ],
    mesh=plsc.VectorSubcoreMesh(core_axis_name="c", subcore_axis_name="s"),
    compiler_params=pltpu.CompilerParams(needs_layout_passes=False),
)(_sc_body)


def kernel(coordinates, w):
    coords_flat = coordinates.T.reshape(-1)
    w_pad = jnp.pad(w, ((0, 0), (0, _DPAD - _NDIM)))
    out_pad = _launch(coords_flat, w_pad)
    return out_pad[:, :_NDIM]
